# baseline (speedup 1.0000x reference)
import functools
import math

import jax
import jax.numpy as jnp
import numpy as np
from jax import lax
from jax.experimental import pallas as pl
from jax.experimental.pallas import tpu as pltpu
from jax.experimental.pallas import tpu_sc as plsc

_BATCH = 1024
_SEQ = 200
_EMB = 64
_FLAT = _BATCH * _SEQ
_NW = 32
_PER_W = _FLAT // _NW
_CHUNK = 160
_NCHUNK = _PER_W // _CHUNK
_NPAIR = _NCHUNK // 2
_SCALE = math.sqrt(_EMB)
_PEROWS = _SEQ + _CHUNK


def _pe_tiled():
    position = np.arange(_SEQ, dtype=np.float32)[:, None]
    div_term = np.exp(
        np.arange(0, _EMB, 2, dtype=np.float32) * (-math.log(10000.0) / _EMB))
    pe = np.zeros((_SEQ, _EMB), dtype=np.float32)
    pe[:, 0::2] = np.sin(position * div_term)
    pe[:, 1::2] = np.cos(position * div_term)
    return np.tile(pe, (2, 1))[:_PEROWS]


_PE2 = _pe_tiled()

_GDN = lax.GatherDimensionNumbers(
    offset_dims=(), collapsed_slice_dims=(0,), start_index_map=(0,))


def _splat(vec, u):
    lane = jnp.full((16, 1), u, jnp.int32)
    return lax.gather(vec, lane, _GDN, (1,),
                      mode=lax.GatherScatterMode.PROMISE_IN_BOUNDS)


def _compute_chunk(buf, obuf, off, idx_v, mask_v, pe_v):
    pos0 = lax.rem(off, _SEQ)

    def row_block(i, carry):
        r0 = i * 16
        m16 = mask_v[pl.ds(off + r0, 16)]
        h16 = jnp.bitwise_and(idx_v[pl.ds(off + r0, 16)], 1).astype(jnp.float32)
        for u in range(16):
            r = r0 + u
            m = _splat(m16, u)
            h = _splat(h16, u)
            ms = m * _SCALE
            for j in range(_EMB // 16):
                lo = buf[r, pl.ds(j * 16, 16)]
                hi = buf[r, pl.ds(_EMB + j * 16, 16)]
                v = lo + h * (hi - lo)
                obuf[i * 8 + u // 2, pl.ds((u % 2) * _EMB + j * 16, 16)] = (
                    v * ms + pe_v[pos0 + r, pl.ds(j * 16, 16)] * m)
        return carry

    lax.fori_loop(0, _CHUNK // 16, row_block, 0)


def _body(tablep, xflat, mflat, pe2, out,
          idx_v, idx2_v, mask_v, pe_v, buf0, buf1, obuf0, obuf1,
          g0, g1, o0, o1):
    nc = 2
    wid = lax.axis_index("s") * nc + lax.axis_index("c")
    base = wid * _PER_W

    pltpu.sync_copy(xflat.at[pl.ds(base, _PER_W)], idx_v)
    pltpu.sync_copy(mflat.at[pl.ds(base, _PER_W)], mask_v)
    pltpu.sync_copy(pe2, pe_v)

    def shift_block(i, carry):
        sl = pl.ds(i * 16, 16)
        idx2_v[sl] = jax.lax.shift_right_logical(idx_v[sl], 1)
        return carry

    lax.fori_loop(0, _PER_W // 16, shift_block, 0)

    pltpu.async_copy(tablep.at[idx2_v.at[pl.ds(0, _CHUNK)]], buf0, g0)

    out_sl = lambda off: out.at[
        pl.ds(pl.multiple_of((base + off) // 2, 8), _CHUNK // 2)]

    def pair(k, carry):
        off0 = 2 * k * _CHUNK
        off1 = off0 + _CHUNK
        off2 = off0 + 2 * _CHUNK

        pltpu.make_async_copy(tablep.at[pl.ds(0, _CHUNK)], buf0, g0).wait()

        @pl.when(k > 0)
        def _():
            pltpu.make_async_copy(obuf1, out_sl(0), o1).wait()

        pltpu.async_copy(tablep.at[idx2_v.at[pl.ds(off1, _CHUNK)]], buf1, g1)

        _compute_chunk(buf0, obuf0, off0, idx_v, mask_v, pe_v)
        pltpu.async_copy(obuf0, out_sl(off0), o0)

        pltpu.make_async_copy(tablep.at[pl.ds(0, _CHUNK)], buf1, g1).wait()
        pltpu.make_async_copy(obuf0, out_sl(0), o0).wait()

        @pl.when(k < _NPAIR - 1)
        def _():
            pltpu.async_copy(
                tablep.at[idx2_v.at[pl.ds(off2, _CHUNK)]], buf0, g0)

        _compute_chunk(buf1, obuf1, off1, idx_v, mask_v, pe_v)
        pltpu.async_copy(obuf1, out_sl(off1), o1)
        return carry

    lax.fori_loop(0, _NPAIR, pair, 0)
    pltpu.make_async_copy(obuf1, out_sl(0), o1).wait()


_emb_lookup = pl.kernel(
    _body,
    out_type=jax.ShapeDtypeStruct((_FLAT // 2, 2 * _EMB), jnp.float32),
    mesh=plsc.VectorSubcoreMesh(core_axis_name="c", subcore_axis_name="s"),
    scratch_types=[
        pltpu.VMEM((_PER_W,), jnp.int32),
        pltpu.VMEM((_PER_W,), jnp.int32),
        pltpu.VMEM((_PER_W,), jnp.float32),
        pltpu.VMEM((_PEROWS, _EMB), jnp.float32),
        pltpu.VMEM((_CHUNK, 2 * _EMB), jnp.float32),
        pltpu.VMEM((_CHUNK, 2 * _EMB), jnp.float32),
        pltpu.VMEM((_CHUNK // 2, 2 * _EMB), jnp.float32),
        pltpu.VMEM((_CHUNK // 2, 2 * _EMB), jnp.float32),
        pltpu.SemaphoreType.DMA,
        pltpu.SemaphoreType.DMA,
        pltpu.SemaphoreType.DMA,
        pltpu.SemaphoreType.DMA,
    ],
)


@jax.jit
def kernel(x, attention_mask, table):
    tablep = table.reshape(_FLAT // _FLAT * 500000, 2 * _EMB)
    xflat = x.reshape(_FLAT)
    mflat = attention_mask.reshape(_FLAT)
    pe2 = jnp.asarray(_PE2)
    out = _emb_lookup(tablep, xflat, mflat, pe2)
    return out.reshape(_BATCH, _SEQ, _EMB)


# speedup vs baseline: 1.4887x; 1.4887x over previous
import functools
import math

import jax
import jax.numpy as jnp
import numpy as np
from jax import lax
from jax.experimental import pallas as pl
from jax.experimental.pallas import tpu as pltpu
from jax.experimental.pallas import tpu_sc as plsc

_BATCH = 1024
_SEQ = 200
_EMB = 64
_FLAT = _BATCH * _SEQ
_NW = 32
_PER_W = _FLAT // _NW
_CHUNK = 160
_NCHUNK = _PER_W // _CHUNK
_NPAIR = _NCHUNK // 2
_SCALE = math.sqrt(_EMB)
_PEROWS = _SEQ + _CHUNK


def _pe_tiled():
    position = np.arange(_SEQ, dtype=np.float32)[:, None]
    div_term = np.exp(
        np.arange(0, _EMB, 2, dtype=np.float32) * (-math.log(10000.0) / _EMB))
    pe = np.zeros((_SEQ, _EMB), dtype=np.float32)
    pe[:, 0::2] = np.sin(position * div_term)
    pe[:, 1::2] = np.cos(position * div_term)
    return np.tile(pe, (2, 1))[:_PEROWS]


_PE2 = _pe_tiled()

_GDN = lax.GatherDimensionNumbers(
    offset_dims=(), collapsed_slice_dims=(0,), start_index_map=(0,))


def _splat(vec, u):
    lane = jnp.full((16, 1), u, jnp.int32)
    return lax.gather(vec, lane, _GDN, (1,),
                      mode=lax.GatherScatterMode.PROMISE_IN_BOUNDS)


def _compute_chunk(buf, obuf, off, idx_v, mask_v, pe_v):
    pos0 = lax.rem(off, _SEQ)

    def row_block(i, carry):
        r0 = i * 16
        m16 = mask_v[pl.ds(off + r0, 16)]
        h16 = (1 - jax.lax.shift_right_logical(
            idx_v[pl.ds(off + r0, 16)] - _SPLIT, 31)).astype(jnp.float32)
        for u in range(16):
            r = r0 + u
            m = _splat(m16, u)
            h = _splat(h16, u)
            ms = m * _SCALE
            for j in range(_EMB // 16):
                lo = buf[r, pl.ds(j * 16, 16)]
                hi = buf[r, pl.ds(_EMB + j * 16, 16)]
                v = lo + h * (hi - lo)
                obuf[i * 8 + u // 2, pl.ds((u % 2) * _EMB + j * 16, 16)] = (
                    v * ms + pe_v[pos0 + r, pl.ds(j * 16, 16)] * m)
        return carry

    lax.fori_loop(0, _CHUNK // 16, row_block, 0)


def _body(tablep, xflat, mflat, pe2, out,
          idx_v, idx2_v, mask_v, pe_v, buf0, buf1, obuf0, obuf1,
          g0, g1, o0, o1):
    nc = 2
    wid = lax.axis_index("s") * nc + lax.axis_index("c")
    base = wid * _PER_W

    pltpu.sync_copy(xflat.at[pl.ds(base, _PER_W)], idx_v)
    pltpu.sync_copy(mflat.at[pl.ds(base, _PER_W)], mask_v)
    pltpu.sync_copy(pe2, pe_v)

    def shift_block(i, carry):
        sl = pl.ds(i * 16, 16)
        v = idx_v[sl]
        hi = 1 - jax.lax.shift_right_logical(v - _SPLIT, 31)
        idx2_v[sl] = v - hi * _SPLIT
        return carry

    lax.fori_loop(0, _PER_W // 16, shift_block, 0)

    pltpu.async_copy(tablep.at[idx2_v.at[pl.ds(0, _CHUNK)]], buf0, g0)

    out_sl = lambda off: out.at[
        pl.ds(pl.multiple_of((base + off) // 2, 8), _CHUNK // 2)]

    def pair(k, carry):
        off0 = 2 * k * _CHUNK
        off1 = off0 + _CHUNK
        off2 = off0 + 2 * _CHUNK

        pltpu.make_async_copy(tablep.at[pl.ds(0, _CHUNK)], buf0, g0).wait()

        @pl.when(k > 0)
        def _():
            pltpu.make_async_copy(obuf1, out_sl(0), o1).wait()

        pltpu.async_copy(tablep.at[idx2_v.at[pl.ds(off1, _CHUNK)]], buf1, g1)

        _compute_chunk(buf0, obuf0, off0, idx_v, mask_v, pe_v)
        pltpu.async_copy(obuf0, out_sl(off0), o0)

        pltpu.make_async_copy(tablep.at[pl.ds(0, _CHUNK)], buf1, g1).wait()
        pltpu.make_async_copy(obuf0, out_sl(0), o0).wait()

        @pl.when(k < _NPAIR - 1)
        def _():
            pltpu.async_copy(
                tablep.at[idx2_v.at[pl.ds(off2, _CHUNK)]], buf0, g0)

        _compute_chunk(buf1, obuf1, off1, idx_v, mask_v, pe_v)
        pltpu.async_copy(obuf1, out_sl(off1), o1)
        return carry

    lax.fori_loop(0, _NPAIR, pair, 0)
    pltpu.make_async_copy(obuf1, out_sl(0), o1).wait()


_VB = 4096
_NBLK = 124
_SPLIT = _NBLK * _VB


def _pack_body(xlo_ref, xhi_ref, y_ref):
    y_ref[...] = jnp.concatenate(
        [jnp.transpose(xlo_ref[...]), jnp.transpose(xhi_ref[...])], axis=1)


_tc_pack = pl.pallas_call(
    _pack_body,
    grid=(_NBLK,),
    in_specs=[
        pl.BlockSpec((_EMB, _VB), lambda i: (0, i)),
        pl.BlockSpec(
            (_EMB, _VB),
            lambda i: (0, jnp.minimum(i + _NBLK, (1000000 - 1) // _VB)),
        ),
    ],
    out_specs=pl.BlockSpec((_VB, 2 * _EMB), lambda i: (i, 0)),
    out_shape=jax.ShapeDtypeStruct((_SPLIT, 2 * _EMB), jnp.float32),
)


_emb_lookup = pl.kernel(
    _body,
    out_type=jax.ShapeDtypeStruct((_FLAT // 2, 2 * _EMB), jnp.float32),
    mesh=plsc.VectorSubcoreMesh(core_axis_name="c", subcore_axis_name="s"),
    scratch_types=[
        pltpu.VMEM((_PER_W,), jnp.int32),
        pltpu.VMEM((_PER_W,), jnp.int32),
        pltpu.VMEM((_PER_W,), jnp.float32),
        pltpu.VMEM((_PEROWS, _EMB), jnp.float32),
        pltpu.VMEM((_CHUNK, 2 * _EMB), jnp.float32),
        pltpu.VMEM((_CHUNK, 2 * _EMB), jnp.float32),
        pltpu.VMEM((_CHUNK // 2, 2 * _EMB), jnp.float32),
        pltpu.VMEM((_CHUNK // 2, 2 * _EMB), jnp.float32),
        pltpu.SemaphoreType.DMA,
        pltpu.SemaphoreType.DMA,
        pltpu.SemaphoreType.DMA,
        pltpu.SemaphoreType.DMA,
    ],
)


@jax.jit
def kernel(x, attention_mask, table):
    tablet = jnp.transpose(table)
    tablep = _tc_pack(tablet, tablet)
    xflat = x.reshape(_FLAT)
    mflat = attention_mask.reshape(_FLAT)
    pe2 = jnp.asarray(_PE2)
    out = _emb_lookup(tablep, xflat, mflat, pe2)
    return out.reshape(_BATCH, _SEQ, _EMB)
